# Initial kernel scaffold; baseline (speedup 1.0000x reference)
import jax
import jax.numpy as jnp
from jax.experimental import pallas as pl


def kernel(x, edge_index, batch, W_pre, b_pre, g_pre, be_pre, Wl0, Wr0, bl0, Wl, Wr, bl, bn_g, bn_b, W_post, b_post, g_post, be_post, W_ro, b_ro):
    raise NotImplementedError("write your pallas kernel here")


# speedup vs baseline: 4.8022x; 4.8022x over previous
import functools

import jax
import jax.numpy as jnp
from jax import lax
from jax.experimental import pallas as pl
from jax.experimental.pallas import tpu as pltpu
from jax.experimental.pallas import tpu_sc as plsc

N = 10000
E = 320000
F_IN = 128
NHID = 32
H = 64
G = 128
C = 10

NC = 2
NS = 16
NW = NC * NS
EDGES_PER_W = E // NW
CHUNK = 80
NCHUNK = EDGES_PER_W // CHUNK
ROWS_PER_TILE = 624
ROWS_REM = N - NS * ROWS_PER_TILE
DEG_W = 16


def _sliced_copy(s, src_at, dst_at):
    r0 = s * ROWS_PER_TILE
    pltpu.sync_copy(src_at(r0, ROWS_PER_TILE), dst_at(r0, ROWS_PER_TILE))

    @pl.when(s == NS - 1)
    def _():
        t0 = NS * ROWS_PER_TILE
        pltpu.sync_copy(src_at(t0, ROWS_REM), dst_at(t0, ROWS_REM))

@functools.lru_cache(maxsize=None)
def _make_agg(D):
    mesh = plsc.VectorSubcoreMesh(core_axis_name="c", subcore_axis_name="s")

    @functools.partial(
        pl.kernel,
        out_type=jax.ShapeDtypeStruct((NC, N, D), jnp.float32),
        mesh=mesh,
        compiler_params=pltpu.CompilerParams(use_tc_tiling_on_sc=False),
        scratch_types=[
            pltpu.VMEM((CHUNK,), jnp.int32),
            pltpu.VMEM((CHUNK,), jnp.int32),
            pltpu.VMEM((CHUNK, D), jnp.float32),
            pltpu.VMEM_SHARED((N, D), jnp.float32),
            pltpu.SemaphoreType.DMA,
        ],
    )
    def agg(h_hbm, src_hbm, dst_hbm, zero_hbm, out_hbm, sidx, didx, rows, acc, sem):
        c = lax.axis_index("c")
        s = lax.axis_index("s")
        w = c * NS + s
        _sliced_copy(s, lambda r, n: zero_hbm.at[pl.ds(r, n)],
                     lambda r, n: acc.at[pl.ds(r, n)])
        plsc.subcore_barrier()
        base = w * EDGES_PER_W

        def body(j, carry):
            off = base + j * CHUNK
            pltpu.sync_copy(src_hbm.at[pl.ds(off, CHUNK)], sidx)
            pltpu.sync_copy(dst_hbm.at[pl.ds(off, CHUNK)], didx)
            pltpu.async_copy(h_hbm.at[sidx], rows, sem).wait()
            pltpu.sync_copy(rows, acc.at[didx], add=True)
            return carry

        lax.fori_loop(0, NCHUNK, body, 0)
        plsc.subcore_barrier()
        _sliced_copy(s, lambda r, n: acc.at[pl.ds(r, n)],
                     lambda r, n: out_hbm.at[c, pl.ds(r, n)])

    return agg


@functools.lru_cache(maxsize=None)
def _make_deg():
    mesh = plsc.VectorSubcoreMesh(core_axis_name="c", subcore_axis_name="s")

    @functools.partial(
        pl.kernel,
        out_type=jax.ShapeDtypeStruct((NC, N, DEG_W), jnp.float32),
        mesh=mesh,
        compiler_params=pltpu.CompilerParams(use_tc_tiling_on_sc=False),
        scratch_types=[
            pltpu.VMEM((CHUNK,), jnp.int32),
            pltpu.VMEM((CHUNK, DEG_W), jnp.float32),
            pltpu.VMEM_SHARED((N, DEG_W), jnp.float32),
            pltpu.SemaphoreType.DMA,
        ],
    )
    def deg(dst_hbm, ones_hbm, zero_hbm, out_hbm, didx, ones_v, acc, sem):
        c = lax.axis_index("c")
        s = lax.axis_index("s")
        w = c * NS + s
        _sliced_copy(s, lambda r, n: zero_hbm.at[pl.ds(r, n)],
                     lambda r, n: acc.at[pl.ds(r, n)])
        pltpu.sync_copy(ones_hbm, ones_v)
        plsc.subcore_barrier()
        base = w * EDGES_PER_W

        def body(j, carry):
            off = base + j * CHUNK
            pltpu.sync_copy(dst_hbm.at[pl.ds(off, CHUNK)], didx)
            pltpu.sync_copy(ones_v, acc.at[didx], add=True)
            return carry

        lax.fori_loop(0, NCHUNK, body, 0)
        plsc.subcore_barrier()
        _sliced_copy(s, lambda r, n: acc.at[pl.ds(r, n)],
                     lambda r, n: out_hbm.at[c, pl.ds(r, n)])

    return deg


POOL_ROWS = 312
POOL_CHUNK = 104
POOL_REM = N - NW * POOL_ROWS


@functools.lru_cache(maxsize=None)
def _make_pool():
    mesh = plsc.VectorSubcoreMesh(core_axis_name="c", subcore_axis_name="s")

    @functools.partial(
        pl.kernel,
        out_type=jax.ShapeDtypeStruct((NC, G, H), jnp.float32),
        mesh=mesh,
        compiler_params=pltpu.CompilerParams(use_tc_tiling_on_sc=False),
        scratch_types=[
            pltpu.VMEM((POOL_CHUNK,), jnp.int32),
            pltpu.VMEM((POOL_CHUNK, H), jnp.float32),
            pltpu.VMEM((POOL_REM,), jnp.int32),
            pltpu.VMEM((POOL_REM, H), jnp.float32),
            pltpu.VMEM_SHARED((G, H), jnp.float32),
            pltpu.SemaphoreType.DMA,
        ],
    )
    def pool(h_hbm, batch_hbm, zero_hbm, out_hbm, bidx, hbuf, tidx, tbuf, acc, sem):
        c = lax.axis_index("c")
        s = lax.axis_index("s")
        w = c * NS + s
        g0 = s * (G // NS)
        pltpu.sync_copy(zero_hbm.at[pl.ds(g0, G // NS)],
                        acc.at[pl.ds(g0, G // NS)])
        plsc.subcore_barrier()
        base = w * POOL_ROWS

        def body(j, carry):
            off = base + j * POOL_CHUNK
            pltpu.sync_copy(batch_hbm.at[pl.ds(off, POOL_CHUNK)], bidx)
            pltpu.sync_copy(h_hbm.at[pl.ds(off, POOL_CHUNK)], hbuf)
            pltpu.sync_copy(hbuf, acc.at[bidx], add=True)
            return carry

        lax.fori_loop(0, POOL_ROWS // POOL_CHUNK, body, 0)

        @pl.when(w == NW - 1)
        def _():
            off = NW * POOL_ROWS
            pltpu.sync_copy(batch_hbm.at[pl.ds(off, POOL_REM)], tidx)
            pltpu.sync_copy(h_hbm.at[pl.ds(off, POOL_REM)], tbuf)
            pltpu.sync_copy(tbuf, acc.at[tidx], add=True)

        plsc.subcore_barrier()
        pltpu.sync_copy(acc.at[pl.ds(g0, G // NS)],
                        out_hbm.at[c, pl.ds(g0, G // NS)])

    return pool


def _bn_relu(h, g, b):
    m = jnp.mean(h, axis=0, keepdims=True)
    v = jnp.mean((h - m) ** 2, axis=0, keepdims=True)
    return jnp.maximum((h - m) / jnp.sqrt(v + 1e-5) * g + b, 0.0)


def _pre_body(x_ref, w_ref, b_ref, g_ref, be_ref, o_ref):
    h = lax.dot_general(x_ref[...], w_ref[...], (((1,), (0,)), ((), ())),
                        preferred_element_type=jnp.float32) + b_ref[...]
    o_ref[...] = _bn_relu(h, g_ref[...], be_ref[...])


def _upd_body(h_ref, p_ref, dp_ref, wl_ref, wr_ref, b_ref, g_ref, be_ref, o_ref):
    deg = dp_ref[0, :, 0:1] + dp_ref[1, :, 0:1]
    deg = jnp.maximum(deg, 1.0)
    mean = (p_ref[0] + p_ref[1]) / deg
    z = (lax.dot_general(mean, wl_ref[...], (((1,), (0,)), ((), ())),
                         preferred_element_type=jnp.float32)
         + lax.dot_general(h_ref[...], wr_ref[...], (((1,), (0,)), ((), ())),
                           preferred_element_type=jnp.float32)
         + b_ref[...])
    o_ref[...] = _bn_relu(z, g_ref[...], be_ref[...])


def _ro_body(pp_ref, wp_ref, bp_ref, g_ref, be_ref, wr_ref, br_ref, o_ref):
    pooled = pp_ref[0] + pp_ref[1]
    hh = jnp.maximum(
        lax.dot_general(pooled, wp_ref[...], (((1,), (0,)), ((), ())),
                        preferred_element_type=jnp.float32) + bp_ref[...], 0.0)
    hh = _bn_relu(hh, g_ref[...], be_ref[...])
    o_ref[...] = (lax.dot_general(hh, wr_ref[...], (((1,), (0,)), ((), ())),
                                  preferred_element_type=jnp.float32)
                  + br_ref[...])


def kernel(x, edge_index, batch, W_pre, b_pre, g_pre, be_pre, Wl0, Wr0, bl0,
           Wl, Wr, bl, bn_g, bn_b, W_post, b_post, g_post, be_post, W_ro, b_ro):
    src = edge_index[0]
    dst = edge_index[1]
    zeros32 = jnp.zeros((N, NHID), jnp.float32)
    zeros64 = jnp.zeros((N, H), jnp.float32)
    zeros_d = jnp.zeros((N, DEG_W), jnp.float32)
    ones_d = jnp.ones((CHUNK, DEG_W), jnp.float32)

    degp = _make_deg()(dst, ones_d, zeros_d)

    h0 = pl.pallas_call(
        _pre_body, out_shape=jax.ShapeDtypeStruct((N, NHID), jnp.float32),
    )(x, W_pre.T, b_pre[None], g_pre[None], be_pre[None])

    def upd(h, parts, wl_, wr_, b_, g_, be_):
        return pl.pallas_call(
            _upd_body, out_shape=jax.ShapeDtypeStruct((N, H), jnp.float32),
        )(h, parts, degp, wl_.T, wr_.T, b_[None], g_[None], be_[None])

    p = _make_agg(NHID)(h0, src, dst, zeros32)
    h = upd(h0, p, Wl0, Wr0, bl0, bn_g[0], bn_b[0])
    for i in range(4):
        p = _make_agg(H)(h, src, dst, zeros64)
        h = upd(h, p, Wl[i], Wr[i], bl[i], bn_g[i + 1], bn_b[i + 1])

    zeros_g = jnp.zeros((G, H), jnp.float32)
    pooled_p = _make_pool()(h, batch, zeros_g)
    out = pl.pallas_call(
        _ro_body, out_shape=jax.ShapeDtypeStruct((G, C), jnp.float32),
    )(pooled_p, W_post.T, b_post[None], g_post[None], be_post[None],
      W_ro.T, b_ro[None])
    return out


# speedup vs baseline: 4.8086x; 1.0013x over previous
import functools

import jax
import jax.numpy as jnp
from jax import lax
from jax.experimental import pallas as pl
from jax.experimental.pallas import tpu as pltpu
from jax.experimental.pallas import tpu_sc as plsc

N = 10000
E = 320000
F_IN = 128
NHID = 32
H = 64
G = 128
C = 10

NC = 2
NS = 16
NW = NC * NS
EDGES_PER_W = E // NW
CHUNK = 80
NCHUNK = EDGES_PER_W // CHUNK
ROWS_PER_TILE = 624
ROWS_REM = N - NS * ROWS_PER_TILE
DEG_W = 16


def _sliced_copy(s, src_at, dst_at):
    r0 = s * ROWS_PER_TILE
    pltpu.sync_copy(src_at(r0, ROWS_PER_TILE), dst_at(r0, ROWS_PER_TILE))

    @pl.when(s == NS - 1)
    def _():
        t0 = NS * ROWS_PER_TILE
        pltpu.sync_copy(src_at(t0, ROWS_REM), dst_at(t0, ROWS_REM))

@functools.lru_cache(maxsize=None)
def _make_agg(D):
    mesh = plsc.VectorSubcoreMesh(core_axis_name="c", subcore_axis_name="s")

    @functools.partial(
        pl.kernel,
        out_type=jax.ShapeDtypeStruct((NC, N, D), jnp.float32),
        mesh=mesh,
        compiler_params=pltpu.CompilerParams(use_tc_tiling_on_sc=False),
        scratch_types=[
            pltpu.VMEM((CHUNK,), jnp.int32),
            pltpu.VMEM((CHUNK,), jnp.int32),
            pltpu.VMEM((CHUNK, D), jnp.float32),
            pltpu.VMEM_SHARED((N, D), jnp.float32),
            pltpu.SemaphoreType.DMA,
        ],
    )
    def agg(h_hbm, src_hbm, dst_hbm, zero_hbm, out_hbm, sidx, didx, rows, acc, sem):
        c = lax.axis_index("c")
        s = lax.axis_index("s")
        w = c * NS + s
        _sliced_copy(s, lambda r, n: zero_hbm.at[pl.ds(r, n)],
                     lambda r, n: acc.at[pl.ds(r, n)])
        plsc.subcore_barrier()
        base = w * EDGES_PER_W

        def body(j, carry):
            off = base + j * CHUNK
            pltpu.sync_copy(src_hbm.at[pl.ds(off, CHUNK)], sidx)
            pltpu.sync_copy(dst_hbm.at[pl.ds(off, CHUNK)], didx)
            pltpu.async_copy(h_hbm.at[sidx], rows, sem).wait()
            pltpu.sync_copy(rows, acc.at[didx], add=True)
            return carry

        lax.fori_loop(0, NCHUNK, body, 0)
        plsc.subcore_barrier()
        _sliced_copy(s, lambda r, n: acc.at[pl.ds(r, n)],
                     lambda r, n: out_hbm.at[c, pl.ds(r, n)])

    return agg


@functools.lru_cache(maxsize=None)
def _make_deg():
    mesh = plsc.VectorSubcoreMesh(core_axis_name="c", subcore_axis_name="s")

    @functools.partial(
        pl.kernel,
        out_type=jax.ShapeDtypeStruct((NC, N, DEG_W), jnp.float32),
        mesh=mesh,
        compiler_params=pltpu.CompilerParams(use_tc_tiling_on_sc=False),
        scratch_types=[
            pltpu.VMEM((CHUNK,), jnp.int32),
            pltpu.VMEM((CHUNK, DEG_W), jnp.float32),
            pltpu.VMEM_SHARED((N, DEG_W), jnp.float32),
            pltpu.SemaphoreType.DMA,
        ],
    )
    def deg(dst_hbm, ones_hbm, zero_hbm, out_hbm, didx, ones_v, acc, sem):
        c = lax.axis_index("c")
        s = lax.axis_index("s")
        w = c * NS + s
        _sliced_copy(s, lambda r, n: zero_hbm.at[pl.ds(r, n)],
                     lambda r, n: acc.at[pl.ds(r, n)])
        pltpu.sync_copy(ones_hbm, ones_v)
        plsc.subcore_barrier()
        base = w * EDGES_PER_W

        def body(j, carry):
            off = base + j * CHUNK
            pltpu.sync_copy(dst_hbm.at[pl.ds(off, CHUNK)], didx)
            pltpu.sync_copy(ones_v, acc.at[didx], add=True)
            return carry

        lax.fori_loop(0, NCHUNK, body, 0)
        plsc.subcore_barrier()
        _sliced_copy(s, lambda r, n: acc.at[pl.ds(r, n)],
                     lambda r, n: out_hbm.at[c, pl.ds(r, n)])

    return deg


POOL_ROWS = 312
POOL_CHUNK = 104
POOL_REM = N - NW * POOL_ROWS


@functools.lru_cache(maxsize=None)
def _make_pool():
    mesh = plsc.VectorSubcoreMesh(core_axis_name="c", subcore_axis_name="s")

    @functools.partial(
        pl.kernel,
        out_type=jax.ShapeDtypeStruct((NC, G, H), jnp.float32),
        mesh=mesh,
        compiler_params=pltpu.CompilerParams(use_tc_tiling_on_sc=False),
        scratch_types=[
            pltpu.VMEM((POOL_CHUNK,), jnp.int32),
            pltpu.VMEM((POOL_CHUNK, H), jnp.float32),
            pltpu.VMEM((POOL_REM,), jnp.int32),
            pltpu.VMEM((POOL_REM, H), jnp.float32),
            pltpu.VMEM_SHARED((G, H), jnp.float32),
            pltpu.SemaphoreType.DMA,
        ],
    )
    def pool(h_hbm, batch_hbm, zero_hbm, out_hbm, bidx, hbuf, tidx, tbuf, acc, sem):
        c = lax.axis_index("c")
        s = lax.axis_index("s")
        w = c * NS + s
        g0 = s * (G // NS)
        pltpu.sync_copy(zero_hbm.at[pl.ds(g0, G // NS)],
                        acc.at[pl.ds(g0, G // NS)])
        plsc.subcore_barrier()
        base = w * POOL_ROWS

        def body(j, carry):
            off = base + j * POOL_CHUNK
            pltpu.sync_copy(batch_hbm.at[pl.ds(off, POOL_CHUNK)], bidx)
            pltpu.sync_copy(h_hbm.at[pl.ds(off, POOL_CHUNK)], hbuf)
            pltpu.sync_copy(hbuf, acc.at[bidx], add=True)
            return carry

        lax.fori_loop(0, POOL_ROWS // POOL_CHUNK, body, 0)

        @pl.when(w == NW - 1)
        def _():
            off = NW * POOL_ROWS
            pltpu.sync_copy(batch_hbm.at[pl.ds(off, POOL_REM)], tidx)
            pltpu.sync_copy(h_hbm.at[pl.ds(off, POOL_REM)], tbuf)
            pltpu.sync_copy(tbuf, acc.at[tidx], add=True)

        plsc.subcore_barrier()
        pltpu.sync_copy(acc.at[pl.ds(g0, G // NS)],
                        out_hbm.at[c, pl.ds(g0, G // NS)])

    return pool


def _dot(a, b):
    return lax.dot_general(a, b, (((1,), (0,)), ((), ())),
                           preferred_element_type=jnp.float32)


def _bn_relu(h, g, b):
    m = jnp.mean(h, axis=0, keepdims=True)
    v = jnp.mean((h - m) ** 2, axis=0, keepdims=True)
    return jnp.maximum((h - m) / jnp.sqrt(v + 1e-5) * g + b, 0.0)


def _pre_body(x_ref, w_ref, b_ref, g_ref, be_ref, o_ref):
    h = _dot(x_ref[...], w_ref[...]) + b_ref[...]
    o_ref[...] = _bn_relu(h, g_ref[...], be_ref[...])


def _upd_body(h_ref, p_ref, dp_ref, wl_ref, wr_ref, b_ref, g_ref, be_ref, o_ref):
    deg = jnp.maximum(dp_ref[0, :, 0:1] + dp_ref[1, :, 0:1], 1.0)
    mean = (p_ref[0] + p_ref[1]) / deg
    z = (_dot(mean, wl_ref[...]) + b_ref[...]) + _dot(h_ref[...], wr_ref[...])
    o_ref[...] = _bn_relu(z, g_ref[...], be_ref[...])


def _ro_body(pp_ref, wp_ref, bp_ref, g_ref, be_ref, wr_ref, br_ref, o_ref):
    pooled = pp_ref[0] + pp_ref[1]
    hh = jnp.maximum(_dot(pooled, wp_ref[...]) + bp_ref[...], 0.0)
    hh = _bn_relu(hh, g_ref[...], be_ref[...])
    o_ref[...] = _dot(hh, wr_ref[...]) + br_ref[...]


def kernel(x, edge_index, batch, W_pre, b_pre, g_pre, be_pre, Wl0, Wr0, bl0,
           Wl, Wr, bl, bn_g, bn_b, W_post, b_post, g_post, be_post, W_ro, b_ro):
    src = edge_index[0]
    dst = edge_index[1]
    zeros32 = jnp.zeros((N, NHID), jnp.float32)
    zeros64 = jnp.zeros((N, H), jnp.float32)
    zeros_d = jnp.zeros((N, DEG_W), jnp.float32)
    ones_d = jnp.ones((CHUNK, DEG_W), jnp.float32)

    degp = _make_deg()(dst, ones_d, zeros_d)

    h0 = pl.pallas_call(
        _pre_body, out_shape=jax.ShapeDtypeStruct((N, NHID), jnp.float32),
    )(x, W_pre.T, b_pre[None], g_pre[None], be_pre[None])

    def upd(h, parts, wl_, wr_, b_, g_, be_):
        return pl.pallas_call(
            _upd_body, out_shape=jax.ShapeDtypeStruct((N, H), jnp.float32),
        )(h, parts, degp, wl_.T, wr_.T, b_[None], g_[None], be_[None])

    p = _make_agg(NHID)(h0, src, dst, zeros32)
    h = upd(h0, p, Wl0, Wr0, bl0, bn_g[0], bn_b[0])
    for i in range(4):
        p = _make_agg(H)(h, src, dst, zeros64)
        h = upd(h, p, Wl[i], Wr[i], bl[i], bn_g[i + 1], bn_b[i + 1])

    zeros_g = jnp.zeros((G, H), jnp.float32)
    pooled_p = _make_pool()(h, batch, zeros_g)
    out = pl.pallas_call(
        _ro_body, out_shape=jax.ShapeDtypeStruct((G, C), jnp.float32),
    )(pooled_p, W_post.T, b_post[None], g_post[None], be_post[None],
      W_ro.T, b_ro[None])
    return out
